# Initial kernel scaffold; baseline (speedup 1.0000x reference)
import jax
import jax.numpy as jnp
from jax.experimental import pallas as pl


def kernel(x, affine_weight, affine_bias):
    raise NotImplementedError("write your pallas kernel here")


# speedup vs baseline: 2.9249x; 2.9249x over previous
import numpy as np

import jax
import jax.numpy as jnp
from jax import lax
from jax.experimental import pallas as pl
from jax.experimental.pallas import tpu as pltpu

_EPS = 1e-5
_HI = lax.Precision.HIGHEST


def _fold_matrices(C, lanes=128):
    f = (np.arange(lanes)[:, None] % C == np.arange(C)[None, :]).astype(np.float32)
    return jnp.asarray(f), jnp.asarray(f.T)


def _norm_kernel(x_ref, w_ref, b_ref, f_ref, ft_ref, y_ref, mean_ref, std_ref,
                 *, inv_t):
    x = x_ref[...]
    s = jnp.sum(x, axis=1)
    sq = jnp.sum(x * x, axis=1)
    f = f_ref[...]
    mean = jnp.dot(s, f, precision=_HI,
                   preferred_element_type=jnp.float32) * inv_t
    msq = jnp.dot(sq, f, precision=_HI,
                  preferred_element_type=jnp.float32) * inv_t
    var = msq - mean * mean
    std = jnp.sqrt(var + _EPS)
    scale = w_ref[...] / std
    shift = b_ref[...] - mean * scale
    ft = ft_ref[...]
    scale_l = jnp.dot(scale, ft, precision=_HI,
                      preferred_element_type=jnp.float32)
    shift_l = jnp.dot(shift, ft, precision=_HI,
                      preferred_element_type=jnp.float32)
    y_ref[...] = x * scale_l[:, None, :] + shift_l[:, None, :]
    mean_ref[...] = mean
    std_ref[...] = std


def kernel(x, affine_weight, affine_bias):
    B, T, C = x.shape
    L = T * C
    lanes = 128
    assert L % lanes == 0 and lanes % C == 0
    G = L // lanes
    inv_t = float(1.0 / T)

    xg = x.reshape(B, G, lanes)
    f, ft = _fold_matrices(C, lanes)
    w2 = affine_weight.astype(jnp.float32).reshape(1, C)
    b2 = affine_bias.astype(jnp.float32).reshape(1, C)

    bb = 256
    while B % bb != 0:
        bb //= 2
    grid = (B // bb,)

    import functools
    body = functools.partial(_norm_kernel, inv_t=inv_t)

    y, mean, std = pl.pallas_call(
        body,
        out_shape=(jax.ShapeDtypeStruct((B, G, lanes), x.dtype),
                   jax.ShapeDtypeStruct((B, C), jnp.float32),
                   jax.ShapeDtypeStruct((B, C), jnp.float32)),
        grid=grid,
        in_specs=[
            pl.BlockSpec((bb, G, lanes), lambda i: (i, 0, 0)),
            pl.BlockSpec((1, C), lambda i: (0, 0)),
            pl.BlockSpec((1, C), lambda i: (0, 0)),
            pl.BlockSpec((lanes, C), lambda i: (0, 0)),
            pl.BlockSpec((C, lanes), lambda i: (0, 0)),
        ],
        out_specs=[
            pl.BlockSpec((bb, G, lanes), lambda i: (i, 0, 0)),
            pl.BlockSpec((bb, C), lambda i: (i, 0)),
            pl.BlockSpec((bb, C), lambda i: (i, 0)),
        ],
        compiler_params=pltpu.CompilerParams(
            dimension_semantics=("parallel",),
            vmem_limit_bytes=48 << 20,
        ),
    )(xg, w2, b2, f, ft)

    return y.reshape(B, T, C), mean.reshape(B, 1, C), std.reshape(B, 1, C)
